# baseline (speedup 1.0000x reference)
import functools
from typing import Sequence

import jax
import jax.numpy as jnp
from jax.experimental import pallas as pl
from jax.experimental.pallas import tpu as pltpu

EPS = 1e-5
SA1_CFG = [(0.1, 16, [32, 32, 64]), (0.2, 32, [64, 64, 128]), (0.4, 128, [64, 96, 128])]
SA2_CFG = [(0.2, 32, [64, 64, 128]), (0.4, 64, [128, 128, 256]), (0.8, 128, [128, 128, 256])]


def _fps(coords, num_centroids):
    B, _, N = coords.shape
    def step(carry, _):
        dists, farthest = carry
        centroid = jnp.take_along_axis(
            coords, jnp.broadcast_to(farthest[:, None, None], (B, 3, 1)), axis=2)
        d = jnp.sum((coords - centroid) ** 2, axis=1)
        dists = jnp.minimum(dists, d)
        nxt = jnp.argmax(dists, axis=1).astype(jnp.int32)
        return (dists, nxt), farthest
    init = (jnp.full((B, N), 1e10, dtype=coords.dtype), jnp.zeros((B,), dtype=jnp.int32))
    _, idx = jax.lax.scan(step, init, None, length=num_centroids)
    return jnp.transpose(idx)


def _ball_query(coords, centroids, radius, max_samples):
    N = coords.shape[2]
    d2 = jnp.sum((centroids[:, :, :, None] - coords[:, :, None, :]) ** 2, axis=1)
    mask = d2 <= radius * radius
    cand = jnp.where(mask, jnp.arange(N, dtype=jnp.int32)[None, None, :], N)
    sidx = jnp.sort(cand, axis=2)[:, :, :max_samples]
    first = sidx[:, :, :1]
    return jnp.where(sidx == N, first, sidx)


def _conv_bn_relu2d(x, W, b):
    y = jnp.einsum('oc,bcsk->bosk', W, x) + b[None, :, None, None]
    m = jnp.mean(y, axis=(0, 2, 3), keepdims=True)
    v = jnp.var(y, axis=(0, 2, 3), keepdims=True)
    return jax.nn.relu((y - m) / jnp.sqrt(v + EPS))


def _set_abstraction(coords, feats, centroid_idx, radius, max_samples, mlp_params):
    B, _, N = coords.shape
    S = centroid_idx.shape[1]
    ci = jnp.broadcast_to(centroid_idx[:, None, :], (B, 3, S))
    centroids = jnp.take_along_axis(coords, ci, axis=2)
    gi = _ball_query(coords, centroids, radius, max_samples)
    gflat = gi.reshape(B, S * max_samples)
    gc = jnp.take_along_axis(coords, jnp.broadcast_to(gflat[:, None, :], (B, 3, S * max_samples)), axis=2)
    gc = gc.reshape(B, 3, S, max_samples) - centroids[:, :, :, None]
    if feats is None:
        cat = gc
    else:
        C = feats.shape[1]
        gf = jnp.take_along_axis(feats, jnp.broadcast_to(gflat[:, None, :], (B, C, S * max_samples)), axis=2)
        cat = jnp.concatenate([gc, gf.reshape(B, C, S, max_samples)], axis=1)
    for (W, b) in mlp_params:
        cat = _conv_bn_relu2d(cat, W, b)
    return jnp.max(cat, axis=-1)


def _global_abstraction(coords, feats, mlp_params):
    cat = coords if feats is None else jnp.concatenate([coords, feats], axis=1)
    for (W, b) in mlp_params:
        y = jnp.einsum('oc,bcn->bon', W, cat) + b[None, :, None]
        m = jnp.mean(y, axis=(0, 2), keepdims=True)
        v = jnp.var(y, axis=(0, 2), keepdims=True)
        cat = jax.nn.relu((y - m) / jnp.sqrt(v + EPS))
    return jnp.max(cat, axis=-1)


def _head_kernel(x_ref, w0_ref, b0_ref, w1_ref, b1_ref, w2_ref, b2_ref, o_ref):
    x = x_ref[...]
    for (w_ref, b_ref, last) in ((w0_ref, b0_ref, False), (w1_ref, b1_ref, False),
                                 (w2_ref, b2_ref, True)):
        x = jnp.dot(x, w_ref[...].T, preferred_element_type=jnp.float32) + b_ref[...][None, :]
        if not last:
            m = jnp.mean(x, axis=0, keepdims=True)
            v = jnp.mean((x - m) ** 2, axis=0, keepdims=True)
            x = jax.nn.relu((x - m) / jnp.sqrt(v + EPS))
    o_ref[...] = x


def _head(x, fc_params):
    (w0, b0), (w1, b1), (w2, b2) = fc_params
    return pl.pallas_call(
        _head_kernel,
        out_shape=jax.ShapeDtypeStruct((x.shape[0], w2.shape[0]), jnp.float32),
    )(x, w0, b0, w1, b1, w2, b2)


def kernel(x, params):
    coords = x[:, :3, :]
    feats = None if x.shape[1] == 3 else x[:, 3:, :]
    B = coords.shape[0]
    c1 = _fps(coords, 512)
    f1 = jnp.concatenate(
        [_set_abstraction(coords, feats, c1, r, k, p)
         for (r, k, _), p in zip(SA1_CFG, params['sa1'])], axis=1)
    coords1 = jnp.take_along_axis(coords, jnp.broadcast_to(c1[:, None, :], (B, 3, 512)), axis=2)
    c2 = _fps(coords1, 128)
    f2 = jnp.concatenate(
        [_set_abstraction(coords1, f1, c2, r, k, p)
         for (r, k, _), p in zip(SA2_CFG, params['sa2'])], axis=1)
    coords2 = jnp.take_along_axis(coords1, jnp.broadcast_to(c2[:, None, :], (B, 3, 128)), axis=2)
    f3 = _global_abstraction(coords2, f2, params['sa3'])
    return _head(f3, params['fc'])


# speedup vs baseline: 1.0027x; 1.0027x over previous
import functools
from typing import Sequence

import jax
import jax.numpy as jnp
from jax import lax
from jax.experimental import pallas as pl
from jax.experimental.pallas import tpu as pltpu
from jax.experimental.pallas import tpu_sc as plsc

EPS = 1e-5
SA1_CFG = [(0.1, 16, [32, 32, 64]), (0.2, 32, [64, 64, 128]), (0.4, 128, [64, 96, 128])]
SA2_CFG = [(0.2, 32, [64, 64, 128]), (0.4, 64, [128, 128, 256]), (0.8, 128, [128, 128, 256])]


def _fps_body(S, cx_ref, cy_ref, cz_ref, idx_ref, dists_ref):
    B, N = cx_ref.shape
    dists_ref[...] = jnp.full((B, N), 1e10, dtype=jnp.float32)
    idx_ref[...] = jnp.zeros((B, S), jnp.int32)
    iota_n = jax.lax.broadcasted_iota(jnp.int32, (B, N), 1)
    iota_s = jax.lax.broadcasted_iota(jnp.int32, (B, S), 1)
    cx = cx_ref[...]
    cy = cy_ref[...]
    cz = cz_ref[...]

    def step(t, farthest):
        oh = (iota_n == farthest).astype(jnp.float32)
        ctrx = jnp.sum(cx * oh, axis=1, keepdims=True)
        ctry = jnp.sum(cy * oh, axis=1, keepdims=True)
        ctrz = jnp.sum(cz * oh, axis=1, keepdims=True)
        d = (cx - ctrx) ** 2 + (cy - ctry) ** 2 + (cz - ctrz) ** 2
        dists = jnp.minimum(dists_ref[...], d)
        dists_ref[...] = dists
        m = jnp.max(dists, axis=1, keepdims=True)
        nxt = jnp.min(jnp.where(dists == m, iota_n, N), axis=1, keepdims=True)
        idx_ref[...] = idx_ref[...] + jnp.where(iota_s == t, farthest, 0)
        return nxt

    jax.lax.fori_loop(0, S, step, jnp.zeros((B, 1), jnp.int32))


def _fps(coords, num_centroids):
    B, _, N = coords.shape
    S = num_centroids
    return pl.pallas_call(
        functools.partial(_fps_body, S),
        out_shape=jax.ShapeDtypeStruct((B, S), jnp.int32),
        scratch_shapes=[pltpu.VMEM((B, N), jnp.float32)],
    )(coords[:, 0, :], coords[:, 1, :], coords[:, 2, :])


def _ballq_sc(coords, cidx, radii, ks):
    B, _, N = coords.shape
    S = cidx.shape[1]
    NC, NS = 2, 16
    NW = NC * NS
    WPB = NW // B
    SW = S // WPB
    r2s = [float(r) * float(r) for r in radii]
    L = 16

    out_type = ([jax.ShapeDtypeStruct((B, S, k), jnp.int32) for k in ks]
                + [jax.ShapeDtypeStruct((B, S), jnp.float32)] * 3)
    scratch = ([pltpu.VMEM((N,), jnp.float32)] * 3
               + [pltpu.VMEM((SW,), jnp.int32)]
               + [pltpu.VMEM((SW,), jnp.float32)] * 3
               + [pltpu.VMEM((SW, k), jnp.int32) for k in ks])

    @functools.partial(
        pl.kernel, out_type=out_type,
        mesh=plsc.VectorSubcoreMesh(core_axis_name="c", subcore_axis_name="s"),
        scratch_types=scratch,
        compiler_params=pltpu.CompilerParams(needs_layout_passes=False))
    def ballq(cx_h, cy_h, cz_h, ci_h, gi0_h, gi1_h, gi2_h, ctx_h, cty_h, ctz_h,
              cxv, cyv, czv, civ, ctxv, ctyv, ctzv, g0, g1, g2):
        wid = lax.axis_index("s") * NC + lax.axis_index("c")
        b = wid // WPB
        s0 = (wid % WPB) * SW
        pltpu.sync_copy(cx_h.at[b], cxv)
        pltpu.sync_copy(cy_h.at[b], cyv)
        pltpu.sync_copy(cz_h.at[b], czv)
        pltpu.sync_copy(ci_h.at[b, pl.ds(s0, SW)], civ)
        iota16 = lax.iota(jnp.int32, L)
        for j in range(SW // L):
            ci16 = civ[pl.ds(j * L, L)]
            ctxv[pl.ds(j * L, L)] = plsc.load_gather(cxv, [ci16])
            ctyv[pl.ds(j * L, L)] = plsc.load_gather(cyv, [ci16])
            ctzv[pl.ds(j * L, L)] = plsc.load_gather(czv, [ci16])

        grefs = (g0, g1, g2)

        def per_centroid(i, _):
            rowv = jnp.full((L,), i, jnp.int32)
            cbx = plsc.load_gather(ctxv, [rowv])
            cby = plsc.load_gather(ctyv, [rowv])
            cbz = plsc.load_gather(ctzv, [rowv])

            def per_chunk(j, cnts):
                base = j * L
                pidx = base + iota16
                px = plsc.load_gather(cxv, [pidx])
                py = plsc.load_gather(cyv, [pidx])
                pz = plsc.load_gather(czv, [pidx])
                dx = px - cbx
                dy = py - cby
                dz = pz - cbz
                d2 = dx * dx + dy * dy + dz * dz
                mask2 = d2 <= r2s[2]
                pos2 = plsc.cumsum(jnp.where(mask2, 1, 0))
                n2 = jnp.max(pos2)

                def do_branches(cnts):
                    new = []
                    for r, (gref, k, r2) in enumerate(zip(grefs, ks, r2s)):
                        if r == 2:
                            mask, pos, np_ = mask2, pos2, n2
                        else:
                            mask = d2 <= r2
                            pos = plsc.cumsum(jnp.where(mask, 1, 0))
                            np_ = jnp.max(pos)
                        wpos = cnts[r] + pos - 1
                        wmask = mask & (wpos < k)
                        plsc.store_scatter(gref, [rowv, wpos], pidx, mask=wmask)
                        new.append(cnts[r] + np_)
                    return tuple(new)

                return lax.cond(n2 > 0, do_branches, lambda c: c, cnts)

            z = jnp.zeros((), jnp.int32)
            cnts = lax.fori_loop(0, N // L, per_chunk, (z, z, z))
            zero16 = jnp.zeros((L,), jnp.int32)
            for r, (gref, k) in enumerate(zip(grefs, ks)):
                cntc = jnp.minimum(cnts[r], k)
                first = plsc.load_gather(gref, [rowv, zero16])
                for jj in range(k // L):
                    idxv = iota16 + jj * L
                    cur = plsc.load_gather(gref, [rowv, idxv])
                    plsc.store_scatter(gref, [rowv, idxv],
                                       jnp.where(idxv < cntc, cur, first))
            return 0

        lax.fori_loop(0, SW, per_centroid, 0)
        pltpu.sync_copy(g0, gi0_h.at[b, pl.ds(s0, SW)])
        pltpu.sync_copy(g1, gi1_h.at[b, pl.ds(s0, SW)])
        pltpu.sync_copy(g2, gi2_h.at[b, pl.ds(s0, SW)])
        pltpu.sync_copy(ctxv, ctx_h.at[b, pl.ds(s0, SW)])
        pltpu.sync_copy(ctyv, cty_h.at[b, pl.ds(s0, SW)])
        pltpu.sync_copy(ctzv, ctz_h.at[b, pl.ds(s0, SW)])

    gi0, gi1, gi2, ctx, cty, ctz = ballq(
        coords[:, 0, :], coords[:, 1, :], coords[:, 2, :], cidx)
    return [gi0, gi1, gi2], (ctx, cty, ctz)


def _ball_query(coords, centroids, radius, max_samples):
    N = coords.shape[2]
    d2 = jnp.sum((centroids[:, :, :, None] - coords[:, :, None, :]) ** 2, axis=1)
    mask = d2 <= radius * radius
    cand = jnp.where(mask, jnp.arange(N, dtype=jnp.int32)[None, None, :], N)
    sidx = jnp.sort(cand, axis=2)[:, :, :max_samples]
    first = sidx[:, :, :1]
    return jnp.where(sidx == N, first, sidx)


def _conv_bn_relu2d(x, W, b):
    y = jnp.einsum('oc,bcsk->bosk', W, x) + b[None, :, None, None]
    m = jnp.mean(y, axis=(0, 2, 3), keepdims=True)
    v = jnp.var(y, axis=(0, 2, 3), keepdims=True)
    return jax.nn.relu((y - m) / jnp.sqrt(v + EPS))


def _set_abstraction(coords, feats, centroids, gi, mlp_params):
    B, _, N = coords.shape
    S = centroids.shape[2]
    max_samples = gi.shape[2]
    gflat = gi.reshape(B, S * max_samples)
    gc = jnp.take_along_axis(coords, jnp.broadcast_to(gflat[:, None, :], (B, 3, S * max_samples)), axis=2)
    gc = gc.reshape(B, 3, S, max_samples) - centroids[:, :, :, None]
    if feats is None:
        cat = gc
    else:
        C = feats.shape[1]
        gf = jnp.take_along_axis(feats, jnp.broadcast_to(gflat[:, None, :], (B, C, S * max_samples)), axis=2)
        cat = jnp.concatenate([gc, gf.reshape(B, C, S, max_samples)], axis=1)
    for (W, b) in mlp_params:
        cat = _conv_bn_relu2d(cat, W, b)
    return jnp.max(cat, axis=-1)


def _global_abstraction(coords, feats, mlp_params):
    cat = coords if feats is None else jnp.concatenate([coords, feats], axis=1)
    for (W, b) in mlp_params:
        y = jnp.einsum('oc,bcn->bon', W, cat) + b[None, :, None]
        m = jnp.mean(y, axis=(0, 2), keepdims=True)
        v = jnp.var(y, axis=(0, 2), keepdims=True)
        cat = jax.nn.relu((y - m) / jnp.sqrt(v + EPS))
    return jnp.max(cat, axis=-1)


def _head_kernel(x_ref, w0_ref, b0_ref, w1_ref, b1_ref, w2_ref, b2_ref, o_ref):
    x = x_ref[...]
    for (w_ref, b_ref, last) in ((w0_ref, b0_ref, False), (w1_ref, b1_ref, False),
                                 (w2_ref, b2_ref, True)):
        x = jnp.dot(x, w_ref[...].T, preferred_element_type=jnp.float32) + b_ref[...][None, :]
        if not last:
            m = jnp.mean(x, axis=0, keepdims=True)
            v = jnp.mean((x - m) ** 2, axis=0, keepdims=True)
            x = jax.nn.relu((x - m) / jnp.sqrt(v + EPS))
    o_ref[...] = x


def _head(x, fc_params):
    (w0, b0), (w1, b1), (w2, b2) = fc_params
    return pl.pallas_call(
        _head_kernel,
        out_shape=jax.ShapeDtypeStruct((x.shape[0], w2.shape[0]), jnp.float32),
    )(x, w0, b0, w1, b1, w2, b2)


def kernel(x, params):
    coords = x[:, :3, :]
    feats = None if x.shape[1] == 3 else x[:, 3:, :]
    B = coords.shape[0]
    c1 = _fps(coords, 512)
    gis1, (ctx1, cty1, ctz1) = _ballq_sc(
        coords, c1, [r for (r, _, _) in SA1_CFG], [k for (_, k, _) in SA1_CFG])
    coords1 = jnp.stack([ctx1, cty1, ctz1], axis=1)
    f1 = jnp.concatenate(
        [_set_abstraction(coords, feats, coords1, gi, p)
         for gi, p in zip(gis1, params['sa1'])], axis=1)
    c2 = _fps(coords1, 128)
    gis2, (ctx2, cty2, ctz2) = _ballq_sc(
        coords1, c2, [r for (r, _, _) in SA2_CFG], [k for (_, k, _) in SA2_CFG])
    coords2 = jnp.stack([ctx2, cty2, ctz2], axis=1)
    f2 = jnp.concatenate(
        [_set_abstraction(coords1, f1, coords2, gi, p)
         for gi, p in zip(gis2, params['sa2'])], axis=1)
    f3 = _global_abstraction(coords2, f2, params['sa3'])
    return _head(f3, params['fc'])


# speedup vs baseline: 196.4851x; 195.9655x over previous
import functools
from typing import Sequence

import jax
import jax.numpy as jnp
from jax import lax
from jax.experimental import pallas as pl
from jax.experimental.pallas import tpu as pltpu
from jax.experimental.pallas import tpu_sc as plsc

EPS = 1e-5
SA1_CFG = [(0.1, 16, [32, 32, 64]), (0.2, 32, [64, 64, 128]), (0.4, 128, [64, 96, 128])]
SA2_CFG = [(0.2, 32, [64, 64, 128]), (0.4, 64, [128, 128, 256]), (0.8, 128, [128, 128, 256])]


def _fps_body(S, cx_ref, cy_ref, cz_ref, idx_ref, dists_ref):
    B, N = cx_ref.shape
    dists_ref[...] = jnp.full((B, N), 1e10, dtype=jnp.float32)
    idx_ref[...] = jnp.zeros((B, S), jnp.int32)
    iota_n = jax.lax.broadcasted_iota(jnp.int32, (B, N), 1)
    iota_s = jax.lax.broadcasted_iota(jnp.int32, (B, S), 1)
    cx = cx_ref[...]
    cy = cy_ref[...]
    cz = cz_ref[...]

    def step(t, farthest):
        oh = (iota_n == farthest).astype(jnp.float32)
        ctrx = jnp.sum(cx * oh, axis=1, keepdims=True)
        ctry = jnp.sum(cy * oh, axis=1, keepdims=True)
        ctrz = jnp.sum(cz * oh, axis=1, keepdims=True)
        d = (cx - ctrx) ** 2 + (cy - ctry) ** 2 + (cz - ctrz) ** 2
        dists = jnp.minimum(dists_ref[...], d)
        dists_ref[...] = dists
        m = jnp.max(dists, axis=1, keepdims=True)
        nxt = jnp.min(jnp.where(dists == m, iota_n, N), axis=1, keepdims=True)
        idx_ref[...] = idx_ref[...] + jnp.where(iota_s == t, farthest, 0)
        return nxt

    jax.lax.fori_loop(0, S, step, jnp.zeros((B, 1), jnp.int32))


def _fps(coords, num_centroids):
    B, _, N = coords.shape
    S = num_centroids
    return pl.pallas_call(
        functools.partial(_fps_body, S),
        out_shape=jax.ShapeDtypeStruct((B, S), jnp.int32),
        scratch_shapes=[pltpu.VMEM((B, N), jnp.float32)],
    )(coords[:, 0, :], coords[:, 1, :], coords[:, 2, :])


def _ballq_sc(coords, cidx, radii, ks):
    B, _, N = coords.shape
    S = cidx.shape[1]
    NC, NS = 2, 16
    NW = NC * NS
    WPB = NW // B
    SW = S // WPB
    r2s = [float(r) * float(r) for r in radii]
    L = 16

    out_type = ([jax.ShapeDtypeStruct((B, S, k), jnp.int32) for k in ks]
                + [jax.ShapeDtypeStruct((B, S), jnp.float32)] * 3)
    scratch = ([pltpu.VMEM((N,), jnp.float32)] * 3
               + [pltpu.VMEM((SW,), jnp.int32)]
               + [pltpu.VMEM((SW,), jnp.float32)] * 3
               + [pltpu.VMEM((SW, k), jnp.int32) for k in ks])

    @functools.partial(
        pl.kernel, out_type=out_type,
        mesh=plsc.VectorSubcoreMesh(core_axis_name="c", subcore_axis_name="s"),
        scratch_types=scratch,
        compiler_params=pltpu.CompilerParams(needs_layout_passes=False))
    def ballq(cx_h, cy_h, cz_h, ci_h, gi0_h, gi1_h, gi2_h, ctx_h, cty_h, ctz_h,
              cxv, cyv, czv, civ, ctxv, ctyv, ctzv, g0, g1, g2):
        wid = lax.axis_index("s") * NC + lax.axis_index("c")
        b = wid // WPB
        s0 = (wid % WPB) * SW
        pltpu.sync_copy(cx_h.at[b], cxv)
        pltpu.sync_copy(cy_h.at[b], cyv)
        pltpu.sync_copy(cz_h.at[b], czv)
        pltpu.sync_copy(ci_h.at[b, pl.ds(s0, SW)], civ)
        iota16 = lax.iota(jnp.int32, L)
        for j in range(SW // L):
            ci16 = civ[pl.ds(j * L, L)]
            ctxv[pl.ds(j * L, L)] = plsc.load_gather(cxv, [ci16])
            ctyv[pl.ds(j * L, L)] = plsc.load_gather(cyv, [ci16])
            ctzv[pl.ds(j * L, L)] = plsc.load_gather(czv, [ci16])

        grefs = (g0, g1, g2)

        def per_centroid(i, _):
            rowv = jnp.full((L,), i, jnp.int32)
            cbx = plsc.load_gather(ctxv, [rowv])
            cby = plsc.load_gather(ctyv, [rowv])
            cbz = plsc.load_gather(ctzv, [rowv])

            def per_chunk(j, cnts):
                base = j * L
                pidx = base + iota16
                px = plsc.load_gather(cxv, [pidx])
                py = plsc.load_gather(cyv, [pidx])
                pz = plsc.load_gather(czv, [pidx])
                dx = px - cbx
                dy = py - cby
                dz = pz - cbz
                d2 = dx * dx + dy * dy + dz * dz
                mask2 = d2 <= r2s[2]
                pos2 = plsc.cumsum(jnp.where(mask2, 1, 0))
                n2 = jnp.max(pos2)

                def do_branches(cnts):
                    new = []
                    for r, (gref, k, r2) in enumerate(zip(grefs, ks, r2s)):
                        if r == 2:
                            mask, pos, np_ = mask2, pos2, n2
                        else:
                            mask = d2 <= r2
                            pos = plsc.cumsum(jnp.where(mask, 1, 0))
                            np_ = jnp.max(pos)
                        wpos = cnts[r] + pos - 1
                        wmask = mask & (wpos < k)
                        plsc.store_scatter(gref, [rowv, wpos], pidx, mask=wmask)
                        new.append(cnts[r] + np_)
                    return tuple(new)

                return lax.cond(n2 > 0, do_branches, lambda c: c, cnts)

            z = jnp.zeros((), jnp.int32)
            cnts = lax.fori_loop(0, N // L, per_chunk, (z, z, z))
            zero16 = jnp.zeros((L,), jnp.int32)
            for r, (gref, k) in enumerate(zip(grefs, ks)):
                cntc = jnp.minimum(cnts[r], k)
                first = plsc.load_gather(gref, [rowv, zero16])
                for jj in range(k // L):
                    idxv = iota16 + jj * L
                    cur = plsc.load_gather(gref, [rowv, idxv])
                    plsc.store_scatter(gref, [rowv, idxv],
                                       jnp.where(idxv < cntc, cur, first))
            return 0

        lax.fori_loop(0, SW, per_centroid, 0)
        pltpu.sync_copy(g0, gi0_h.at[b, pl.ds(s0, SW)])
        pltpu.sync_copy(g1, gi1_h.at[b, pl.ds(s0, SW)])
        pltpu.sync_copy(g2, gi2_h.at[b, pl.ds(s0, SW)])
        pltpu.sync_copy(ctxv, ctx_h.at[b, pl.ds(s0, SW)])
        pltpu.sync_copy(ctyv, cty_h.at[b, pl.ds(s0, SW)])
        pltpu.sync_copy(ctzv, ctz_h.at[b, pl.ds(s0, SW)])

    gi0, gi1, gi2, ctx, cty, ctz = ballq(
        coords[:, 0, :], coords[:, 1, :], coords[:, 2, :], cidx)
    return [gi0, gi1, gi2], (ctx, cty, ctz)


def _j_set_abstraction(coords, feats, centroids, gi, mlp_params):
    B, _, N = coords.shape
    S = centroids.shape[2]
    Kk = gi.shape[2]
    gflat = gi.reshape(B, S * Kk)
    gc = jnp.take_along_axis(coords, jnp.broadcast_to(gflat[:, None, :], (B, 3, S * Kk)), axis=2)
    gc = gc.reshape(B, 3, S, Kk) - centroids[:, :, :, None]
    if feats is None:
        cat = gc
    else:
        C = feats.shape[1]
        gf = jnp.take_along_axis(feats, jnp.broadcast_to(gflat[:, None, :], (B, C, S * Kk)), axis=2)
        cat = jnp.concatenate([gc, gf.reshape(B, C, S, Kk)], axis=1)
    for (W, b) in mlp_params:
        y = jnp.einsum('oc,bcsk->bosk', W, cat) + b[None, :, None, None]
        m = jnp.mean(y, axis=(0, 2, 3), keepdims=True)
        v = jnp.var(y, axis=(0, 2, 3), keepdims=True)
        cat = jax.nn.relu((y - m) / jnp.sqrt(v + EPS))
    return jnp.max(cat, axis=-1)


def _j_global_abstraction(coords, feats, mlp_params):
    cat = coords if feats is None else jnp.concatenate([coords, feats], axis=1)
    for (W, b) in mlp_params:
        y = jnp.einsum('oc,bcn->bon', W, cat) + b[None, :, None]
        m = jnp.mean(y, axis=(0, 2), keepdims=True)
        v = jnp.var(y, axis=(0, 2), keepdims=True)
        cat = jax.nn.relu((y - m) / jnp.sqrt(v + EPS))
    return jnp.max(cat, axis=-1)


def _gather_body(S_t, K, N, C, gi_ref, tab_ref, o_ref):
    R = S_t * K
    gi = gi_ref[0]
    oh = (gi[..., None] == jax.lax.broadcasted_iota(jnp.int32, (S_t, K, N), 2)
          ).astype(jnp.float32)
    g = jnp.dot(oh.reshape(R, N), tab_ref[0], preferred_element_type=jnp.float32,
                precision=jax.lax.Precision.HIGHEST)
    o_ref[0] = g.T


def _gather_cat(gi, tab, S_t):
    B, S, K = gi.shape
    N, C = tab.shape[1], tab.shape[2]
    out = pl.pallas_call(
        functools.partial(_gather_body, S_t, K, N, C),
        grid=(B, S // S_t),
        in_specs=[
            pl.BlockSpec((1, S_t, K), lambda b_, s: (b_, s, 0)),
            pl.BlockSpec((1, N, C), lambda b_, s: (b_, 0, 0)),
        ],
        out_specs=pl.BlockSpec((1, C, S_t * K), lambda b_, s: (b_, 0, s)),
        out_shape=jax.ShapeDtypeStruct((B, C, S * K), jnp.float32),
    )(gi, tab)
    return jax.lax.optimization_barrier(out.reshape(B, C, S, K))


def _j_mlp(cat, mlp_params):
    for (W, b) in mlp_params:
        y = jnp.einsum('oc,bcsk->bosk', W, cat) + b[None, :, None, None]
        m = jnp.mean(y, axis=(0, 2, 3), keepdims=True)
        v = jnp.var(y, axis=(0, 2, 3), keepdims=True)
        cat = jax.nn.relu((y - m) / jnp.sqrt(v + EPS))
    return jnp.max(cat, axis=-1)


def _stats_to_affine(stats, count):
    m = stats[0:1, :] / count
    v = stats[1:2, :] / count - m * m
    sc = 1.0 / jnp.sqrt(v + EPS)
    return sc, -m * sc


def _l1_sa1_body(S_t, K, N, gi_ref, tab_ref, ctr_ref, w_ref, b_ref, y_ref, st_ref):
    R = S_t * K
    gi = gi_ref[0]
    lo = jnp.bitwise_and(gi, 127)[..., None]
    hi = jnp.right_shift(gi, 7)[..., None]
    ohlo = (lo == jax.lax.broadcasted_iota(jnp.int32, (S_t, K, 128), 2)).astype(jnp.float32)
    a = jnp.dot(ohlo.reshape(R, 128), tab_ref[0],
                preferred_element_type=jnp.float32,
                precision=jax.lax.Precision.HIGHEST)
    NH = N // 128
    a = a.reshape(S_t, K, NH, 4)
    ohhi = (hi == jax.lax.broadcasted_iota(jnp.int32, (S_t, K, NH), 2)).astype(jnp.float32)
    gath = jnp.sum(a * ohhi[..., None], axis=2)
    rel = (gath[..., :3] - ctr_ref[0][:, None, :]).reshape(R, 3)
    y = jnp.dot(rel, w_ref[...], preferred_element_type=jnp.float32) + b_ref[...]
    y_ref[0] = y
    first = (pl.program_id(0) == 0) & (pl.program_id(1) == 0)
    @pl.when(first)
    def _():
        st_ref[...] = jnp.zeros_like(st_ref)
    st_ref[0:1, :] += jnp.sum(y, axis=0, keepdims=True)
    st_ref[1:2, :] += jnp.sum(y * y, axis=0, keepdims=True)


def _l1_sa1(gi, tab, ctr, W, b, S_t):
    B, S, K = gi.shape
    N = tab.shape[2] // 4 * 128
    C1 = W.shape[0]
    grid = (B, S // S_t)
    return pl.pallas_call(
        functools.partial(_l1_sa1_body, S_t, K, N),
        grid=grid,
        in_specs=[
            pl.BlockSpec((1, S_t, K), lambda b_, s: (b_, s, 0)),
            pl.BlockSpec((1, 128, tab.shape[2]), lambda b_, s: (b_, 0, 0)),
            pl.BlockSpec((1, S_t, 3), lambda b_, s: (b_, s, 0)),
            pl.BlockSpec((3, C1), lambda b_, s: (0, 0)),
            pl.BlockSpec((1, C1), lambda b_, s: (0, 0)),
        ],
        out_specs=[
            pl.BlockSpec((1, S_t * K, C1), lambda b_, s: (b_, s, 0)),
            pl.BlockSpec((8, C1), lambda b_, s: (0, 0)),
        ],
        out_shape=[jax.ShapeDtypeStruct((B, S * K, C1), jnp.float32),
                   jax.ShapeDtypeStruct((8, C1), jnp.float32)],
    )(gi, tab, ctr, W.T, b[None, :])


def _l1_sa2_body(S_t, K, N, gi_ref, z_ref, ctr_ref, wc_ref, b_ref, y_ref, st_ref):
    R = S_t * K
    C = z_ref.shape[2]
    gi = gi_ref[0]
    oh = (gi[..., None] == jax.lax.broadcasted_iota(jnp.int32, (S_t, K, N), 2)
          ).astype(jnp.float32)
    gath = jnp.dot(oh.reshape(R, N), z_ref[0],
                   preferred_element_type=jnp.float32,
                   precision=jax.lax.Precision.HIGHEST)
    off = jnp.dot(ctr_ref[0], wc_ref[...], preferred_element_type=jnp.float32) - b_ref[...]
    y = (gath.reshape(S_t, K, C) - off[:, None, :]).reshape(R, C)
    y_ref[0] = y
    first = (pl.program_id(0) == 0) & (pl.program_id(1) == 0)
    @pl.when(first)
    def _():
        st_ref[...] = jnp.zeros_like(st_ref)
    st_ref[0:1, :] += jnp.sum(y, axis=0, keepdims=True)
    st_ref[1:2, :] += jnp.sum(y * y, axis=0, keepdims=True)


def _l1_sa2(gi, z, ctr, Wc, b, S_t):
    B, S, K = gi.shape
    N = z.shape[1]
    C1 = z.shape[2]
    grid = (B, S // S_t)
    return pl.pallas_call(
        functools.partial(_l1_sa2_body, S_t, K, N),
        grid=grid,
        in_specs=[
            pl.BlockSpec((1, S_t, K), lambda b_, s: (b_, s, 0)),
            pl.BlockSpec((1, N, C1), lambda b_, s: (b_, 0, 0)),
            pl.BlockSpec((1, S_t, 3), lambda b_, s: (b_, s, 0)),
            pl.BlockSpec((3, C1), lambda b_, s: (0, 0)),
            pl.BlockSpec((1, C1), lambda b_, s: (0, 0)),
        ],
        out_specs=[
            pl.BlockSpec((1, S_t * K, C1), lambda b_, s: (b_, s, 0)),
            pl.BlockSpec((8, C1), lambda b_, s: (0, 0)),
        ],
        out_shape=[jax.ShapeDtypeStruct((B, S * K, C1), jnp.float32),
                   jax.ShapeDtypeStruct((8, C1), jnp.float32)],
    )(gi, z, ctr, Wc.T, b[None, :])


def _mid_body(x_ref, sc_ref, sh_ref, w_ref, b_ref, y_ref, st_ref):
    xh = jax.nn.relu(x_ref[...] * sc_ref[...] + sh_ref[...])
    y = jnp.dot(xh, w_ref[...], preferred_element_type=jnp.float32) + b_ref[...]
    y_ref[...] = y
    @pl.when(pl.program_id(0) == 0)
    def _():
        st_ref[...] = jnp.zeros_like(st_ref)
    st_ref[0:1, :] += jnp.sum(y, axis=0, keepdims=True)
    st_ref[1:2, :] += jnp.sum(y * y, axis=0, keepdims=True)


def _mid_layer(x, stats, count, W, b, R_t=4096):
    M, Cin = x.shape
    Cout = W.shape[0]
    sc, sh = _stats_to_affine(stats, count)
    grid = (M // R_t,)
    return pl.pallas_call(
        _mid_body,
        grid=grid,
        in_specs=[
            pl.BlockSpec((R_t, Cin), lambda i: (i, 0)),
            pl.BlockSpec((1, Cin), lambda i: (0, 0)),
            pl.BlockSpec((1, Cin), lambda i: (0, 0)),
            pl.BlockSpec((Cin, Cout), lambda i: (0, 0)),
            pl.BlockSpec((1, Cout), lambda i: (0, 0)),
        ],
        out_specs=[
            pl.BlockSpec((R_t, Cout), lambda i: (i, 0)),
            pl.BlockSpec((8, Cout), lambda i: (0, 0)),
        ],
        out_shape=[jax.ShapeDtypeStruct((M, Cout), jnp.float32),
                   jax.ShapeDtypeStruct((8, Cout), jnp.float32)],
    )(x, sc, sh, W.T, b[None, :])


def _last_body(S_t, K, x_ref, sc_ref, sh_ref, w_ref, b_ref, y_ref, st_ref):
    xh = jax.nn.relu(x_ref[0] * sc_ref[...] + sh_ref[...])
    y = jnp.dot(xh, w_ref[...], preferred_element_type=jnp.float32) + b_ref[...]
    Cout = y.shape[1]
    y_ref[0] = jnp.max(y.reshape(S_t, K, Cout), axis=1)
    first = (pl.program_id(0) == 0) & (pl.program_id(1) == 0)
    @pl.when(first)
    def _():
        st_ref[...] = jnp.zeros_like(st_ref)
    st_ref[0:1, :] += jnp.sum(y, axis=0, keepdims=True)
    st_ref[1:2, :] += jnp.sum(y * y, axis=0, keepdims=True)


def _last_layer(x, stats, count, W, b, B, S, K, S_t):
    Cin = x.shape[1]
    Cout = W.shape[0]
    sc, sh = _stats_to_affine(stats, count)
    grid = (B, S // S_t)
    return pl.pallas_call(
        functools.partial(_last_body, S_t, K),
        grid=grid,
        in_specs=[
            pl.BlockSpec((1, S_t * K, Cin), lambda b_, s: (b_, s, 0)),
            pl.BlockSpec((1, Cin), lambda b_, s: (0, 0)),
            pl.BlockSpec((1, Cin), lambda b_, s: (0, 0)),
            pl.BlockSpec((Cin, Cout), lambda b_, s: (0, 0)),
            pl.BlockSpec((1, Cout), lambda b_, s: (0, 0)),
        ],
        out_specs=[
            pl.BlockSpec((1, S_t, Cout), lambda b_, s: (b_, s, 0)),
            pl.BlockSpec((8, Cout), lambda b_, s: (0, 0)),
        ],
        out_shape=[jax.ShapeDtypeStruct((B, S, Cout), jnp.float32),
                   jax.ShapeDtypeStruct((8, Cout), jnp.float32)],
    )(x.reshape(B, S * K, Cin), sc, sh, W.T, b[None, :])


def _branch_mlp(y1, st1, mlp_params, B, S, K, S_t):
    count = float(B * S * K)
    y, st = y1, st1
    for (W, b) in mlp_params[1:-1]:
        y, st = _mid_layer(y.reshape(B * S * K, -1), st, count, W, b)
    WL, bL = mlp_params[-1]
    return _last_layer(y.reshape(B * S * K, -1), st, count, WL, bL, B, S, K, S_t)


def _z_body(c_ref, f_ref, sc_ref, sh_ref, wc_ref, wf_ref, z_ref):
    xh = jax.nn.relu(f_ref[0] * sc_ref[...] + sh_ref[...])
    z = (jnp.dot(c_ref[0], wc_ref[...], preferred_element_type=jnp.float32)
         + jnp.dot(xh, wf_ref[...], preferred_element_type=jnp.float32))
    z_ref[0] = z


def _z_kernel(c1, f1raw, sc, sh, Wc, Wf):
    B, S, _ = c1.shape
    CF = f1raw.shape[2]
    CZ = Wc.shape[1]
    return pl.pallas_call(
        _z_body,
        grid=(B,),
        in_specs=[
            pl.BlockSpec((1, S, 3), lambda b_: (b_, 0, 0)),
            pl.BlockSpec((1, S, CF), lambda b_: (b_, 0, 0)),
            pl.BlockSpec((1, CF), lambda b_: (0, 0)),
            pl.BlockSpec((1, CF), lambda b_: (0, 0)),
            pl.BlockSpec((3, CZ), lambda b_: (0, 0)),
            pl.BlockSpec((CF, CZ), lambda b_: (0, 0)),
        ],
        out_specs=pl.BlockSpec((1, S, CZ), lambda b_: (b_, 0, 0)),
        out_shape=jax.ShapeDtypeStruct((B, S, CZ), jnp.float32),
    )(c1, f1raw, sc, sh, Wc, Wf)


def _global_body(B, S, c_ref, f_ref, sc_ref, sh_ref,
                 w0c_ref, w0f_ref, b0_ref, w1_ref, b1_ref, w2_ref, b2_ref, o_ref):
    xh = jax.nn.relu(f_ref[...] * sc_ref[...] + sh_ref[...])
    y = (jnp.dot(c_ref[...], w0c_ref[...], preferred_element_type=jnp.float32)
         + jnp.dot(xh, w0f_ref[...], preferred_element_type=jnp.float32)
         + b0_ref[...])
    M = B * S
    for w_ref, b_ref in ((w1_ref, b1_ref), (w2_ref, b2_ref), (None, None)):
        m = jnp.sum(y, axis=0, keepdims=True) / M
        v = jnp.sum(y * y, axis=0, keepdims=True) / M - m * m
        y = jax.nn.relu((y - m) / jnp.sqrt(v + EPS))
        if w_ref is not None:
            y = jnp.dot(y, w_ref[...], preferred_element_type=jnp.float32) + b_ref[...]
    o_ref[...] = jnp.max(y.reshape(B, S, y.shape[1]), axis=1)


def _global_abstraction_k(c2, f2raw, sc, sh, mlp_params):
    B, S, _ = c2.shape
    CF = f2raw.shape[2]
    (W0, b0), (W1, b1), (W2, b2) = mlp_params
    return pl.pallas_call(
        functools.partial(_global_body, B, S),
        out_shape=jax.ShapeDtypeStruct((B, W2.shape[0]), jnp.float32),
    )(c2.reshape(B * S, 3), f2raw.reshape(B * S, CF), sc, sh,
      W0[:, :3].T, W0[:, 3:].T, b0[None, :], W1.T, b1[None, :], W2.T, b2[None, :])


def _head_kernel(x_ref, w0_ref, b0_ref, w1_ref, b1_ref, w2_ref, b2_ref, o_ref):
    x = x_ref[...]
    for (w_ref, b_ref, last) in ((w0_ref, b0_ref, False), (w1_ref, b1_ref, False),
                                 (w2_ref, b2_ref, True)):
        x = jnp.dot(x, w_ref[...].T, preferred_element_type=jnp.float32) + b_ref[...][None, :]
        if not last:
            m = jnp.mean(x, axis=0, keepdims=True)
            v = jnp.mean((x - m) ** 2, axis=0, keepdims=True)
            x = jax.nn.relu((x - m) / jnp.sqrt(v + EPS))
    o_ref[...] = x


def _head(x, fc_params):
    (w0, b0), (w1, b1), (w2, b2) = fc_params
    return pl.pallas_call(
        _head_kernel,
        out_shape=jax.ShapeDtypeStruct((x.shape[0], w2.shape[0]), jnp.float32),
    )(x, w0, b0, w1, b1, w2, b2)


_ST_SA1 = {16: 128, 32: 64, 128: 16}
_ST_SA2 = {32: 64, 64: 32, 128: 16}


_GT_SA1 = {16: 64, 32: 32, 128: 8}
_GT_SA2 = {32: 32, 64: 16, 128: 8}


def kernel(x, params):
    coords = x[:, :3, :]
    B, _, N = coords.shape

    c1 = _fps(coords, 512)
    gis1, (ctx1, cty1, ctz1) = _ballq_sc(
        coords, c1, [r for (r, _, _) in SA1_CFG], [k for (_, k, _) in SA1_CFG])
    coords1 = jnp.stack([ctx1, cty1, ctz1], axis=1)
    ctr1 = jnp.stack([ctx1, cty1, ctz1], axis=2)
    tab1 = jnp.transpose(coords, (0, 2, 1))
    f1_parts = []
    for gi, p, (_, K, _) in zip(gis1, params['sa1'], SA1_CFG):
        g = _gather_cat(gi, tab1, _GT_SA1[K])
        cat = g - coords1[:, :, :, None]
        f1_parts.append(_j_mlp(cat, p))
    f1 = jnp.concatenate(f1_parts, axis=1)

    c2 = _fps(coords1, 128)
    gis2, (ctx2, cty2, ctz2) = _ballq_sc(
        coords1, c2, [r for (r, _, _) in SA2_CFG], [k for (_, k, _) in SA2_CFG])
    ctr2 = jnp.stack([ctx2, cty2, ctz2], axis=2)
    tab2 = jnp.concatenate([ctr1, jnp.transpose(f1, (0, 2, 1))], axis=2)
    coords2 = jnp.stack([ctx2, cty2, ctz2], axis=1)
    f2_parts = []
    for gi, p, (_, K, _) in zip(gis2, params['sa2'], SA2_CFG):
        g = _gather_cat(gi, tab2, _GT_SA2[K])
        gc = g[:, :3] - coords2[:, :, :, None]
        cat = jnp.concatenate([gc, g[:, 3:]], axis=1)
        f2_parts.append(_j_mlp(cat, p))
    f2 = jnp.concatenate(f2_parts, axis=1)

    coords2 = jnp.stack([ctx2, cty2, ctz2], axis=1)
    f3 = _j_global_abstraction(coords2, f2, params['sa3'])
    n = len(params['fc'])
    xh = f3
    for i, (W, b) in enumerate(params['fc']):
        xh = xh @ W.T + b
        if i < n - 1:
            m = jnp.mean(xh, axis=0, keepdims=True)
            v = jnp.var(xh, axis=0, keepdims=True)
            xh = jax.nn.relu((xh - m) / jnp.sqrt(v + EPS))
    return xh
